# baseline (speedup 1.0000x reference)
import functools

import jax
import jax.numpy as jnp
from jax import lax
from jax.experimental import pallas as pl
from jax.experimental.pallas import tpu as pltpu
from jax.experimental.pallas import tpu_sc as plsc

NUM_EDGES = 3_200_000
NUM_NODES = 100_000
NC, NS, L = 2, 16, 16
NW = NC * NS
PER_W = NUM_EDGES // NW
CHUNK = 2_000
NCHUNK = PER_W // CHUNK
NVEC = CHUNK // L

PRE_CHUNK = 1_000
PRE_NCHUNK = NUM_NODES // PRE_CHUNK

_SC_PARAMS = pltpu.CompilerParams(
    needs_layout_passes=False, use_tc_tiling_on_sc=False)


def _make_precompose():
    mesh = plsc.VectorSubcoreMesh(core_axis_name="c", subcore_axis_name="s")
    out_t = tuple(
        jax.ShapeDtypeStruct((NUM_NODES, 8), jnp.float32) for _ in range(2))

    @functools.partial(
        pl.kernel,
        mesh=mesh,
        out_type=out_t,
        compiler_params=_SC_PARAMS,
        scratch_types=[
            pltpu.VMEM((PRE_CHUNK,), jnp.int32),
            pltpu.VMEM((PRE_CHUNK, 8), jnp.float32),
            pltpu.SemaphoreType.DMA,
        ],
    )
    def run(nid_c, nid_g, p_c, p_g, c_c, c_g, nid_v, rows_v, sem):
        wid = lax.axis_index("s") * NC + lax.axis_index("c")

        def body(k, carry):
            cid = wid + NW * k

            @pl.when(cid < PRE_NCHUNK)
            def _():
                base = cid * PRE_CHUNK
                for nid, p, c in ((nid_c, p_c, c_c), (nid_g, p_g, c_g)):
                    pltpu.sync_copy(nid.at[pl.ds(base, PRE_CHUNK)], nid_v)
                    pltpu.async_copy(p.at[nid_v], rows_v, sem).wait()
                    pltpu.sync_copy(rows_v, c.at[pl.ds(base, PRE_CHUNK), :])

            return carry

        lax.fori_loop(0, (PRE_NCHUNK + NW - 1) // NW, body, 0)

    return run


def _make_main():
    mesh = plsc.VectorSubcoreMesh(core_axis_name="c", subcore_axis_name="s")
    out_t = tuple(
        jax.ShapeDtypeStruct((NUM_EDGES,), jnp.float32) for _ in range(6))

    @functools.partial(
        pl.kernel,
        mesh=mesh,
        out_type=out_t,
        compiler_params=_SC_PARAMS,
        scratch_types=[
            pltpu.VMEM((CHUNK,), jnp.int32),
            pltpu.VMEM((CHUNK,), jnp.int32),
            pltpu.VMEM((CHUNK,), jnp.int32),
            pltpu.VMEM((CHUNK,), jnp.int32),
            pltpu.VMEM((CHUNK, 8), jnp.float32),
            pltpu.VMEM((CHUNK, 8), jnp.float32),
            pltpu.VMEM((CHUNK, 8), jnp.float32),
            pltpu.VMEM((CHUNK, 8), jnp.float32),
            pltpu.VMEM((CHUNK,), jnp.float32),
            pltpu.VMEM((CHUNK,), jnp.float32),
            pltpu.VMEM((CHUNK,), jnp.float32),
            pltpu.VMEM((CHUNK,), jnp.float32),
            pltpu.VMEM((CHUNK,), jnp.float32),
            pltpu.VMEM((CHUNK,), jnp.float32),
            pltpu.VMEM((CHUNK,), jnp.float32),
            pltpu.VMEM((CHUNK,), jnp.float32),
            pltpu.VMEM((CHUNK,), jnp.float32),
            pltpu.VMEM((CHUNK,), jnp.float32),
            pltpu.VMEM((CHUNK,), jnp.float32),
            pltpu.VMEM((CHUNK,), jnp.float32),
            pltpu.SemaphoreType.DMA,
            pltpu.SemaphoreType.DMA,
            pltpu.SemaphoreType.DMA,
            pltpu.SemaphoreType.DMA,
            pltpu.SemaphoreType.DMA,
            pltpu.SemaphoreType.DMA,
        ],
    )
    def run(ei, c_c, c_g,
            o_sls, o_sb, o_ss, o_dls, o_db, o_ds,
            i0a_v, i1a_v, i0b_v, i1b_v, r0a_v, r1a_v, r0b_v, r1b_v,
            oa0, oa1, oa2, oa3, oa4, oa5,
            ob0, ob1, ob2, ob3, ob4, ob5,
            sra, srb, sia, sib, ssa, ssb):
        wid = lax.axis_index("s") * NC + lax.axis_index("c")
        idxs = ((i0a_v, i1a_v, sia), (i0b_v, i1b_v, sib))
        rows = ((r0a_v, r1a_v, sra), (r0b_v, r1b_v, srb))
        outs = (((oa0, oa1, oa2, oa3, oa4, oa5), ssa),
                ((ob0, ob1, ob2, ob3, ob4, ob5), ssb))
        outrefs = (o_sls, o_sb, o_ss, o_dls, o_db, o_ds)

        def issue_idx(t, which):
            i0_v, i1_v, si = idxs[which]
            base = wid * PER_W + t * CHUNK
            pltpu.async_copy(ei.at[pl.ds(base, CHUNK)], i0_v, si)
            pltpu.async_copy(ei.at[pl.ds(NUM_EDGES + base, CHUNK)], i1_v, si)

        def drain_idx(which):
            i0_v, i1_v, si = idxs[which]
            pltpu.make_async_copy(ei.at[pl.ds(0, CHUNK)], i0_v, si).wait()
            pltpu.make_async_copy(ei.at[pl.ds(0, CHUNK)], i1_v, si).wait()

        def issue_rows(which):
            i0_v, i1_v, _ = idxs[which]
            r0_v, r1_v, sr = rows[which]
            pltpu.async_copy(c_c.at[i0_v], r0_v, sr)
            pltpu.async_copy(c_g.at[i1_v], r1_v, sr)

        def drain_rows(which):
            i0_v, i1_v, _ = idxs[which]
            r0_v, r1_v, sr = rows[which]
            pltpu.make_async_copy(c_c.at[i0_v], r0_v, sr).wait()
            pltpu.make_async_copy(c_g.at[i1_v], r1_v, sr).wait()

        def unpack(which):
            r0_v, r1_v, _ = rows[which]
            obufs, _ = outs[which]

            def step(i, c2):
                row = i * L + lax.iota(jnp.int32, L)
                for rv, obs in ((r0_v, obufs[:3]), (r1_v, obufs[3:])):
                    for col, buf in enumerate(obs):
                        cvec = jnp.full((L,), col, jnp.int32)
                        buf[pl.ds(i * L, L)] = plsc.load_gather(rv, [row, cvec])
                return c2

            lax.fori_loop(0, NVEC, step, 0, unroll=8)

        def issue_stores(t, which):
            obufs, ss = outs[which]
            base = wid * PER_W + t * CHUNK
            for buf, oref in zip(obufs, outrefs):
                pltpu.async_copy(buf, oref.at[pl.ds(base, CHUNK)], ss)

        def drain_stores(which):
            obufs, ss = outs[which]
            for buf, oref in zip(obufs, outrefs):
                pltpu.make_async_copy(
                    buf, oref.at[pl.ds(0, CHUNK)], ss).wait()

        def step_chunk(t, cur, nxt):
            drain_rows(cur)

            @pl.when(t + 1 < NCHUNK)
            def _():
                drain_idx(nxt)
                issue_rows(nxt)

            @pl.when(t + 2 < NCHUNK)
            def _():
                issue_idx(t + 2, cur)

            @pl.when(t >= 2)
            def _():
                drain_stores(cur)

            unpack(cur)
            issue_stores(t, cur)

        issue_idx(0, 0)
        issue_idx(1, 1)
        drain_idx(0)
        issue_rows(0)

        @pl.loop(0, NCHUNK, step=2)
        def _(t):
            step_chunk(t, 0, 1)
            step_chunk(t + 1, 1, 0)

        drain_stores(0)
        drain_stores(1)

    return run


_PRE = _make_precompose()
_MAIN = _make_main()


def kernel(edge_index, n_id_cell, n_id_gene, logscale_cell, bias_cell,
           std_cell, logscale_gene, bias_gene, std_gene):
    zc = jnp.zeros_like(logscale_cell)
    zg = jnp.zeros_like(logscale_gene)
    p_cell = jnp.stack(
        [logscale_cell, bias_cell, std_cell, zc, zc, zc, zc, zc], axis=1)
    p_gene = jnp.stack(
        [logscale_gene, bias_gene, std_gene, zg, zg, zg, zg, zg], axis=1)
    c_cell, c_gene = _PRE(n_id_cell, n_id_gene, p_cell, p_gene)
    return _MAIN(edge_index.reshape(-1), c_cell, c_gene)


# speedup vs baseline: 1.0027x; 1.0027x over previous
import functools

import jax
import jax.numpy as jnp
from jax import lax
from jax.experimental import pallas as pl
from jax.experimental.pallas import tpu as pltpu
from jax.experimental.pallas import tpu_sc as plsc

NUM_EDGES = 3_200_000
NUM_NODES = 100_000
NC, NS, L = 2, 16, 16
NW = NC * NS
PER_W = NUM_EDGES // NW
CHUNK = 2_000
NCHUNK = PER_W // CHUNK
NVEC = CHUNK // L

PRE_CHUNK = 1_000
PRE_NCHUNK = NUM_NODES // PRE_CHUNK

_SC_PARAMS = pltpu.CompilerParams(
    needs_layout_passes=False, use_tc_tiling_on_sc=False)


def _make_precompose():
    mesh = plsc.VectorSubcoreMesh(core_axis_name="c", subcore_axis_name="s")
    out_t = tuple(
        jax.ShapeDtypeStruct((NUM_NODES, 8), jnp.float32) for _ in range(2))

    @functools.partial(
        pl.kernel,
        mesh=mesh,
        out_type=out_t,
        compiler_params=_SC_PARAMS,
        scratch_types=[
            pltpu.VMEM((PRE_CHUNK,), jnp.int32),
            pltpu.VMEM((PRE_CHUNK, 8), jnp.float32),
            pltpu.SemaphoreType.DMA,
        ],
    )
    def run(nid_c, nid_g, p_c, p_g, c_c, c_g, nid_v, rows_v, sem):
        wid = lax.axis_index("s") * NC + lax.axis_index("c")

        def body(k, carry):
            cid = wid + NW * k

            @pl.when(cid < PRE_NCHUNK)
            def _():
                base = cid * PRE_CHUNK
                for nid, p, c in ((nid_c, p_c, c_c), (nid_g, p_g, c_g)):
                    pltpu.sync_copy(nid.at[pl.ds(base, PRE_CHUNK)], nid_v)
                    pltpu.async_copy(p.at[nid_v], rows_v, sem).wait()
                    pltpu.sync_copy(rows_v, c.at[pl.ds(base, PRE_CHUNK), :])

            return carry

        lax.fori_loop(0, (PRE_NCHUNK + NW - 1) // NW, body, 0)

    return run


def _make_main():
    mesh = plsc.VectorSubcoreMesh(core_axis_name="c", subcore_axis_name="s")
    out_t = tuple(
        jax.ShapeDtypeStruct((NUM_EDGES,), jnp.float32) for _ in range(6))

    @functools.partial(
        pl.kernel,
        mesh=mesh,
        out_type=out_t,
        compiler_params=_SC_PARAMS,
        scratch_types=[
            pltpu.VMEM((CHUNK,), jnp.int32),
            pltpu.VMEM((CHUNK,), jnp.int32),
            pltpu.VMEM((CHUNK,), jnp.int32),
            pltpu.VMEM((CHUNK,), jnp.int32),
            pltpu.VMEM((CHUNK, 8), jnp.float32),
            pltpu.VMEM((CHUNK, 8), jnp.float32),
            pltpu.VMEM((CHUNK, 8), jnp.float32),
            pltpu.VMEM((CHUNK, 8), jnp.float32),
            pltpu.VMEM((CHUNK,), jnp.float32),
            pltpu.VMEM((CHUNK,), jnp.float32),
            pltpu.VMEM((CHUNK,), jnp.float32),
            pltpu.VMEM((CHUNK,), jnp.float32),
            pltpu.VMEM((CHUNK,), jnp.float32),
            pltpu.VMEM((CHUNK,), jnp.float32),
            pltpu.VMEM((CHUNK,), jnp.float32),
            pltpu.VMEM((CHUNK,), jnp.float32),
            pltpu.VMEM((CHUNK,), jnp.float32),
            pltpu.VMEM((CHUNK,), jnp.float32),
            pltpu.VMEM((CHUNK,), jnp.float32),
            pltpu.VMEM((CHUNK,), jnp.float32),
            pltpu.SemaphoreType.DMA,
            pltpu.SemaphoreType.DMA,
            pltpu.SemaphoreType.DMA,
            pltpu.SemaphoreType.DMA,
            pltpu.SemaphoreType.DMA,
            pltpu.SemaphoreType.DMA,
        ],
    )
    def run(ei, c_c, c_g,
            o_sls, o_sb, o_ss, o_dls, o_db, o_ds,
            i0a_v, i1a_v, i0b_v, i1b_v, r0a_v, r1a_v, r0b_v, r1b_v,
            oa0, oa1, oa2, oa3, oa4, oa5,
            ob0, ob1, ob2, ob3, ob4, ob5,
            sra, srb, sia, sib, ssa, ssb):
        wid = lax.axis_index("s") * NC + lax.axis_index("c")
        idxs = ((i0a_v, i1a_v, sia), (i0b_v, i1b_v, sib))
        rows = ((r0a_v, r1a_v, sra), (r0b_v, r1b_v, srb))
        outs = (((oa0, oa1, oa2, oa3, oa4, oa5), ssa),
                ((ob0, ob1, ob2, ob3, ob4, ob5), ssb))
        outrefs = (o_sls, o_sb, o_ss, o_dls, o_db, o_ds)

        def issue_idx(t, which):
            i0_v, i1_v, si = idxs[which]
            base = wid * PER_W + t * CHUNK
            pltpu.async_copy(ei.at[pl.ds(base, CHUNK)], i0_v, si)
            pltpu.async_copy(ei.at[pl.ds(NUM_EDGES + base, CHUNK)], i1_v, si)

        def drain_idx(which):
            i0_v, i1_v, si = idxs[which]
            pltpu.make_async_copy(ei.at[pl.ds(0, CHUNK)], i0_v, si).wait()
            pltpu.make_async_copy(ei.at[pl.ds(0, CHUNK)], i1_v, si).wait()

        def issue_rows(which):
            i0_v, i1_v, _ = idxs[which]
            r0_v, r1_v, sr = rows[which]
            pltpu.async_copy(c_c.at[i0_v], r0_v, sr)
            pltpu.async_copy(c_g.at[i1_v], r1_v, sr)

        def drain_rows(which):
            i0_v, i1_v, _ = idxs[which]
            r0_v, r1_v, sr = rows[which]
            pltpu.make_async_copy(c_c.at[i0_v], r0_v, sr).wait()
            pltpu.make_async_copy(c_g.at[i1_v], r1_v, sr).wait()

        def unpack(which):
            r0_v, r1_v, _ = rows[which]
            obufs, _ = outs[which]

            def step(i, c2):
                row = i * L + lax.iota(jnp.int32, L)
                for rv, obs in ((r0_v, obufs[:3]), (r1_v, obufs[3:])):
                    for col, buf in enumerate(obs):
                        cvec = jnp.full((L,), col, jnp.int32)
                        buf[pl.ds(i * L, L)] = plsc.load_gather(rv, [row, cvec])
                return c2

            lax.fori_loop(0, NVEC, step, 0)

        def issue_stores(t, which):
            obufs, ss = outs[which]
            base = wid * PER_W + t * CHUNK
            for buf, oref in zip(obufs, outrefs):
                pltpu.async_copy(buf, oref.at[pl.ds(base, CHUNK)], ss)

        def drain_stores(which):
            obufs, ss = outs[which]
            for buf, oref in zip(obufs, outrefs):
                pltpu.make_async_copy(
                    buf, oref.at[pl.ds(0, CHUNK)], ss).wait()

        def step_chunk(t, cur, nxt):
            drain_rows(cur)

            @pl.when(t + 1 < NCHUNK)
            def _():
                drain_idx(nxt)
                issue_rows(nxt)

            @pl.when(t + 2 < NCHUNK)
            def _():
                issue_idx(t + 2, cur)

            @pl.when(t >= 2)
            def _():
                drain_stores(cur)

            unpack(cur)
            issue_stores(t, cur)

        issue_idx(0, 0)
        issue_idx(1, 1)
        drain_idx(0)
        issue_rows(0)

        @pl.loop(0, NCHUNK, step=2)
        def _(t):
            step_chunk(t, 0, 1)
            step_chunk(t + 1, 1, 0)

        drain_stores(0)
        drain_stores(1)

    return run


_PRE = _make_precompose()
_MAIN = _make_main()


def kernel(edge_index, n_id_cell, n_id_gene, logscale_cell, bias_cell,
           std_cell, logscale_gene, bias_gene, std_gene):
    zc = jnp.zeros_like(logscale_cell)
    zg = jnp.zeros_like(logscale_gene)
    p_cell = jnp.stack(
        [logscale_cell, bias_cell, std_cell, zc, zc, zc, zc, zc], axis=1)
    p_gene = jnp.stack(
        [logscale_gene, bias_gene, std_gene, zg, zg, zg, zg, zg], axis=1)
    c_cell, c_gene = _PRE(n_id_cell, n_id_gene, p_cell, p_gene)
    return _MAIN(edge_index.reshape(-1), c_cell, c_gene)
